# baseline (speedup 1.0000x reference)
import functools

import jax
import jax.numpy as jnp
from jax import lax
from jax.experimental import pallas as pl
from jax.experimental.pallas import tpu as pltpu
from jax.experimental.pallas import tpu_sc as plsc

INPUT_DIM = 1000000
HALF = INPUT_DIM // 2
D = 32

NC = 2
NS = 16
NW = NC * NS
L = 16

BSZ = 16384
HIST = 50
BW = BSZ // NW
BBLK = 128
NUNITS = HIST * (BW // BBLK)


def _embed_kernel():
    mesh = plsc.VectorSubcoreMesh(core_axis_name="c", subcore_axis_name="s")

    @functools.partial(
        pl.kernel,
        out_type=jax.ShapeDtypeStruct((HIST, D // 8, BSZ // BBLK, 8, BBLK),
                                      jnp.float32),
        mesh=mesh,
        compiler_params=pltpu.CompilerParams(use_tc_tiling_on_sc=False,
                                             needs_layout_passes=False),
        scratch_types=[
            pltpu.VMEM((BW * HIST,), jnp.int32),
            pltpu.VMEM((2, 2 * BBLK, D), jnp.float32),
            pltpu.VMEM((2, D // 8, 8, BBLK), jnp.float32),
            pltpu.VMEM((2, BBLK), jnp.int32),
            pltpu.VMEM((2, BBLK), jnp.int32),
            pltpu.VMEM((2, BBLK), jnp.int32),
            pltpu.SemaphoreType.DMA,
            pltpu.SemaphoreType.DMA,
            pltpu.SemaphoreType.DMA,
        ],
    )
    def k(idx_hbm, e1_hbm, e2_hbm, out_hbm,
          idx_v, rows_v, obuf_v, idx1b, idx2b, rselb, gsem0, gsem1, wsem):
        wid = lax.axis_index("s") * NC + lax.axis_index("c")
        iota = lax.broadcasted_iota(jnp.int32, (L,), 0)
        iota50 = iota * HIST

        pltpu.sync_copy(idx_hbm.at[pl.ds(wid * BW * HIST, BW * HIST)], idx_v)

        def prep(u, slot):
            h = u // (BW // BBLK)
            bblk = u % (BW // BBLK)
            base = bblk * BBLK * HIST + h
            for g in range(BBLK // L):
                ivec = iota50 + (base + g * L * HIST)
                idx16 = plsc.load_gather(idx_v, [ivec])
                flip = idx16 >= HALF
                idx1b[slot, pl.ds(g * L, L)] = jnp.minimum(idx16, HALF - 1)
                idx2b[slot, pl.ds(g * L, L)] = jnp.maximum(idx16 - HALF, 0)
                rselb[slot, pl.ds(g * L, L)] = (
                    (iota + g * L) + jnp.where(flip, BBLK, 0))

        def fire_gathers(slot, sem):
            ga = pltpu.async_copy(
                e1_hbm.at[idx1b.at[slot]],
                rows_v.at[slot, pl.ds(0, BBLK)], sem)
            gb = pltpu.async_copy(
                e2_hbm.at[idx2b.at[slot]],
                rows_v.at[slot, pl.ds(BBLK, BBLK)], sem)
            return ga, gb

        def transpose_select(slot):
            for g in range(BBLK // L):
                rsel16 = rselb[slot, pl.ds(g * L, L)]
                for c in range(D):
                    col = jnp.full((L,), c, jnp.int32)
                    v = plsc.load_gather(rows_v.at[slot], [rsel16, col])
                    obuf_v[slot, c // 8, c % 8, pl.ds(g * L, L)] = v

        def fire_writes(u, slot):
            h = u // (BW // BBLK)
            bblk_g = wid * (BW // BBLK) + (u % (BW // BBLK))
            return [
                pltpu.async_copy(obuf_v.at[slot, cb],
                                 out_hbm.at[h, cb, bblk_g], wsem)
                for cb in range(D // 8)
            ]

        def body(p, carry):
            u0 = p * 2
            u1 = u0 + 1
            prep(u0, 0)
            g0 = fire_gathers(0, gsem0)
            prep(u1, 1)
            g1 = fire_gathers(1, gsem1)
            g0[0].wait()
            g0[1].wait()
            transpose_select(0)
            w0 = fire_writes(u0, 0)
            g1[0].wait()
            g1[1].wait()
            transpose_select(1)
            w1 = fire_writes(u1, 1)
            for w in w0 + w1:
                w.wait()
            return carry

        lax.fori_loop(0, NUNITS // 2, body, 0)

    return k


def kernel(inputs, e1, e2):
    bsz, hist = inputs.shape
    idx = inputs.reshape(bsz * hist).astype(jnp.int32)
    out5 = _embed_kernel()(idx, e1, e2)
    x = out5.transpose(2, 4, 0, 1, 3)
    return x.reshape(bsz, hist, D)
